# baseline (speedup 1.0000x reference)
import functools

import jax
import jax.numpy as jnp
from jax import lax
from jax.experimental import pallas as pl
from jax.experimental.pallas import tpu as pltpu
from jax.experimental.pallas import tpu_sc as plsc

VOCAB = 1000000
EMBED = 32
B_TOTAL = 4096 * 200
NUM_CORES = 2
NUM_SUBCORES = 16
NW = NUM_CORES * NUM_SUBCORES
B_PER_W = B_TOTAL // NW
CHUNK = 1600
N_CHUNKS = B_PER_W // CHUNK

BV = 8192
SUB = BV // 512
NBLK = 123
T4_ROWS = NBLK * BV // 4

_mesh = plsc.VectorSubcoreMesh(core_axis_name="c", subcore_axis_name="s")


def _relayout_body(tt_ref, out_ref):
    t = tt_ref[...].T
    out_ref[...] = jnp.concatenate(
        [jnp.concatenate(
            [t[512 * q + 128 * c:512 * q + 128 * (c + 1), :]
             for c in range(4)], axis=1)
         for q in range(SUB)], axis=0)


_relayout_table = pl.pallas_call(
    _relayout_body,
    grid=(NBLK,),
    in_specs=[pl.BlockSpec((32, BV), lambda i: (0, i))],
    out_specs=pl.BlockSpec((BV // 4, 128), lambda i: (i, 0)),
    out_shape=jax.ShapeDtypeStruct((T4_ROWS, 128), jnp.float32),
)


U_B = 512
U_PER_S = 4096 // U_B
N_UNITS = 200 * U_PER_S
U_PER_W = N_UNITS // NW


@functools.partial(
    pl.kernel,
    mesh=_mesh,
    out_type=jax.ShapeDtypeStruct((200, EMBED, 4096), jnp.float32),
    scratch_types=[
        pltpu.VMEM((2, U_B), jnp.int32),
        pltpu.VMEM((2, U_B, EMBED), jnp.float32),
        pltpu.VMEM((2, EMBED, U_B), jnp.float32),
        pltpu.SemaphoreType.DMA,
        pltpu.SemaphoreType.DMA,
        pltpu.SemaphoreType.DMA,
        pltpu.SemaphoreType.DMA,
        pltpu.SemaphoreType.DMA,
        pltpu.SemaphoreType.DMA,
    ],
    compiler_params=pltpu.CompilerParams(
        use_tc_tiling_on_sc=False, needs_layout_passes=False),
)
def _embed_lookup(idxt_hbm, table_hbm, out_hbm,
                  idx_b, rows_b, slab_b, si0, si1, sg0, sg1, so0, so1):
    wid = lax.axis_index("s") * NUM_CORES + lax.axis_index("c")
    g0 = wid * U_PER_W
    sem_i = (si0, si1)
    sem_g = (sg0, sg1)
    sem_o = (so0, so1)
    lane = lax.iota(jnp.int32, 16)

    def unit_su(u):
        g = g0 + u
        return g // U_PER_S, (g % U_PER_S) * U_B

    def idx_dma(u, h):
        s, b0 = unit_su(u)
        return pltpu.make_async_copy(
            idxt_hbm.at[s, pl.ds(b0, U_B)], idx_b.at[h], sem_i[h])

    def gather(h):
        return pltpu.make_async_copy(
            table_hbm.at[idx_b.at[h]], rows_b.at[h], sem_g[h])

    def store(u, h):
        s, b0 = unit_su(u)
        return pltpu.make_async_copy(
            slab_b.at[h], out_hbm.at[s, :, pl.ds(b0, U_B)],
            sem_o[h])

    def transform(h):
        def body(i, carry):
            v = idx_b[h, pl.ds(i * 16, 16)]
            idx_b[h, pl.ds(i * 16, 16)] = (
                (v & -512) + ((v & 127) << 2) + ((v >> 7) & 3))
            return carry
        lax.fori_loop(0, U_B // 16, body, 0)

    def transpose(h):
        def body(p, carry):
            rowv = lane + 16 * p
            for e in range(EMBED):
                x = plsc.load_gather(
                    rows_b.at[h], [rowv, jnp.full((16,), e, jnp.int32)])
                slab_b[h, e, pl.ds(16 * p, 16)] = x
            return carry
        lax.fori_loop(0, U_B // 16, body, 0)

    def unit_step(u, h, first, last):
        idx_dma(u, h).wait()
        transform(h)
        gather(h).start()
        if not last:
            idx_dma(u + 1, h ^ 1).start()
        gather(h).wait()
        if not first:
            store(u - 2, h).wait()
        transpose(h)
        store(u, h).start()

    idx_dma(0, 0).start()
    unit_step(0, 0, True, False)
    unit_step(1, 1, True, False)

    def main_body(i, carry):
        u = 2 * i
        unit_step(u, 0, False, False)
        unit_step(u + 1, 1, False, False)
        return carry

    lax.fori_loop(1, U_PER_W // 2 - 1, main_body, 0)
    unit_step(U_PER_W - 2, 0, False, False)
    unit_step(U_PER_W - 1, 1, False, True)
    store(U_PER_W - 2, 0).wait()
    store(U_PER_W - 1, 1).wait()


def kernel(inputs, table):
    t4 = _relayout_table(table.T)
    tview = t4.reshape(T4_ROWS * 4, EMBED)
    out3 = _embed_lookup(inputs.T, tview)
    return out3.transpose(2, 0, 1)


# speedup vs baseline: 1.0858x; 1.0858x over previous
import functools

import jax
import jax.numpy as jnp
from jax import lax
from jax.experimental import pallas as pl
from jax.experimental.pallas import tpu as pltpu
from jax.experimental.pallas import tpu_sc as plsc

VOCAB = 1000000
EMBED = 32
B_TOTAL = 4096 * 200
NUM_CORES = 2
NUM_SUBCORES = 16
NW = NUM_CORES * NUM_SUBCORES
B_PER_W = B_TOTAL // NW
CHUNK = 1600
N_CHUNKS = B_PER_W // CHUNK

BV = 8192
SUB = BV // 512
NBLK = 123
T4_ROWS = NBLK * BV // 4

_mesh = plsc.VectorSubcoreMesh(core_axis_name="c", subcore_axis_name="s")


def _relayout_body(tt_ref, out_ref):
    t = tt_ref[...].T
    out_ref[...] = jnp.concatenate(
        [jnp.concatenate(
            [t[512 * q + 128 * c:512 * q + 128 * (c + 1), :]
             for c in range(4)], axis=1)
         for q in range(SUB)], axis=0)


_relayout_table = pl.pallas_call(
    _relayout_body,
    grid=(NBLK,),
    in_specs=[pl.BlockSpec((32, BV), lambda i: (0, i))],
    out_specs=pl.BlockSpec((BV // 4, 128), lambda i: (i, 0)),
    out_shape=jax.ShapeDtypeStruct((T4_ROWS, 128), jnp.float32),
)


U_B = 512
U_PER_S = 4096 // U_B
N_UNITS = 200 * U_PER_S
U_PER_W = N_UNITS // NW


@functools.partial(
    pl.kernel,
    mesh=_mesh,
    out_type=jax.ShapeDtypeStruct((200, EMBED, 4096), jnp.float32),
    scratch_types=[
        pltpu.VMEM((2, U_B), jnp.int32),
        pltpu.VMEM((2, U_B, EMBED), jnp.float32),
        pltpu.VMEM((2, EMBED, U_B), jnp.float32),
        pltpu.SemaphoreType.DMA,
        pltpu.SemaphoreType.DMA,
        pltpu.SemaphoreType.DMA,
        pltpu.SemaphoreType.DMA,
        pltpu.SemaphoreType.DMA,
        pltpu.SemaphoreType.DMA,
    ],
    compiler_params=pltpu.CompilerParams(
        use_tc_tiling_on_sc=False, needs_layout_passes=False),
)
def _embed_lookup(idxt_hbm, table_hbm, out_hbm,
                  idx_b, rows_b, slab_b, si0, si1, sg0, sg1, so0, so1):
    wid = lax.axis_index("s") * NUM_CORES + lax.axis_index("c")
    g0 = wid * U_PER_W
    sem_i = (si0, si1)
    sem_g = (sg0, sg1)
    sem_o = (so0, so1)
    lane = lax.iota(jnp.int32, 16)

    def unit_su(u):
        g = g0 + u
        return g // U_PER_S, (g % U_PER_S) * U_B

    def idx_dma(u, h):
        s, b0 = unit_su(u)
        return pltpu.make_async_copy(
            idxt_hbm.at[s, pl.ds(b0, U_B)], idx_b.at[h], sem_i[h])

    def gather(h):
        return pltpu.make_async_copy(
            table_hbm.at[idx_b.at[h]], rows_b.at[h], sem_g[h])

    def store(u, h):
        s, b0 = unit_su(u)
        return pltpu.make_async_copy(
            slab_b.at[h], out_hbm.at[s, :, pl.ds(b0, U_B)],
            sem_o[h])

    def transform(h):
        def body(i, carry):
            v = idx_b[h, pl.ds(i * 16, 16)]
            idx_b[h, pl.ds(i * 16, 16)] = (
                (v & -512) + ((v & 127) << 2) + ((v >> 7) & 3))
            return carry
        lax.fori_loop(0, U_B // 16, body, 0)

    def transpose(h):
        def body(p, carry):
            rowv = lane + 16 * p
            for e in range(EMBED):
                x = plsc.load_gather(
                    rows_b.at[h], [rowv, jnp.full((16,), e, jnp.int32)])
                slab_b[h, e, pl.ds(16 * p, 16)] = x
            return carry
        lax.fori_loop(0, U_B // 16, body, 0)

    def unit_step(u, h, prefetch_idx, prefetch_gather, first):
        gather(h).wait()
        if prefetch_idx:
            idx_dma(u + 2, h).start()
        if prefetch_gather:
            idx_dma(u + 1, h ^ 1).wait()
            transform(h ^ 1)
            gather(h ^ 1).start()
        if not first:
            store(u - 2, h).wait()
        transpose(h)
        store(u, h).start()

    idx_dma(0, 0).start()
    idx_dma(1, 1).start()
    idx_dma(0, 0).wait()
    transform(0)
    gather(0).start()

    unit_step(0, 0, True, True, True)
    unit_step(1, 1, True, True, True)

    def main_body(i, carry):
        u = 2 * i
        unit_step(u, 0, True, True, False)
        unit_step(u + 1, 1, True, True, False)
        return carry

    lax.fori_loop(1, U_PER_W // 2 - 1, main_body, 0)
    unit_step(U_PER_W - 2, 0, False, True, False)
    unit_step(U_PER_W - 1, 1, False, False, False)
    store(U_PER_W - 2, 0).wait()
    store(U_PER_W - 1, 1).wait()


def kernel(inputs, table):
    t4 = _relayout_table(table.T)
    tview = t4.reshape(T4_ROWS * 4, EMBED)
    out3 = _embed_lookup(inputs.T, tview)
    return out3.transpose(2, 0, 1)


# speedup vs baseline: 2.0777x; 1.9135x over previous
import functools

import jax
import jax.numpy as jnp
from jax import lax
from jax.experimental import pallas as pl
from jax.experimental.pallas import tpu as pltpu
from jax.experimental.pallas import tpu_sc as plsc

VOCAB = 1000000
EMBED = 32
B_TOTAL = 4096 * 200
NUM_CORES = 2
NUM_SUBCORES = 16
NW = NUM_CORES * NUM_SUBCORES
B_PER_W = B_TOTAL // NW
CHUNK = 1600
N_CHUNKS = B_PER_W // CHUNK

BV = 8192
SUB = BV // 512
NBLK = 123
T4_ROWS = NBLK * BV // 4

_mesh = plsc.VectorSubcoreMesh(core_axis_name="c", subcore_axis_name="s")


def _relayout_body(tt_ref, out_ref):
    t = tt_ref[...].T
    out_ref[...] = jnp.concatenate(
        [jnp.concatenate(
            [t[512 * q + 128 * c:512 * q + 128 * (c + 1), :]
             for c in range(4)], axis=1)
         for q in range(SUB)], axis=0)


_relayout_table = pl.pallas_call(
    _relayout_body,
    grid=(NBLK,),
    in_specs=[pl.BlockSpec((32, BV), lambda i: (0, i))],
    out_specs=pl.BlockSpec((BV // 4, 128), lambda i: (i, 0)),
    out_shape=jax.ShapeDtypeStruct((T4_ROWS, 128), jnp.float32),
)


@functools.partial(
    pl.kernel,
    mesh=_mesh,
    out_type=jax.ShapeDtypeStruct((B_TOTAL, EMBED), jnp.float32),
    scratch_types=[
        pltpu.VMEM((B_PER_W,), jnp.int32),
        pltpu.VMEM((CHUNK, EMBED), jnp.float32),
        pltpu.VMEM((CHUNK, EMBED), jnp.float32),
        pltpu.SemaphoreType.DMA,
        pltpu.SemaphoreType.DMA,
        pltpu.SemaphoreType.DMA,
        pltpu.SemaphoreType.DMA,
    ],
    compiler_params=pltpu.CompilerParams(use_tc_tiling_on_sc=False),
)
def _embed_lookup(idx_hbm, table_hbm, out_hbm,
                  idx_v, rows0, rows1, sg0, sg1, so0, so1):
    wid = lax.axis_index("s") * NUM_CORES + lax.axis_index("c")
    base = wid * B_PER_W

    rows_v = (rows0, rows1)
    sem_g = (sg0, sg1)
    sem_o = (so0, so1)

    def transform(j):
        def body(i, carry):
            off = j * CHUNK + i * 16
            v = idx_v[pl.ds(off, 16)]
            r = (v & -512) + ((v & 127) << 2) + ((v >> 7) & 3)
            idx_v[pl.ds(off, 16)] = r
            return carry
        lax.fori_loop(0, CHUNK // 16, body, 0)

    def gather(j, b):
        return pltpu.make_async_copy(
            table_hbm.at[idx_v.at[pl.ds(j * CHUNK, CHUNK)]], rows_v[b], sem_g[b])

    def store(j, b):
        return pltpu.make_async_copy(
            rows_v[b], out_hbm.at[pl.ds(base + j * CHUNK, CHUNK)], sem_o[b])

    pltpu.sync_copy(idx_hbm.at[pl.ds(base, B_PER_W)], idx_v)

    transform(0)
    gather(0, 0).start()
    for j in range(N_CHUNKS):
        b = j & 1
        nb = b ^ 1
        if j + 1 < N_CHUNKS:
            if j >= 1:
                store(j - 1, nb).wait()
            transform(j + 1)
            gather(j + 1, nb).start()
        gather(j, b).wait()
        store(j, b).start()
    store(N_CHUNKS - 2, 0).wait()
    store(N_CHUNKS - 1, 1).wait()


def _txp_body(in_ref, out_ref):
    out_ref[...] = in_ref[...].T


_transpose_out = pl.pallas_call(
    _txp_body,
    grid=(4, 10),
    in_specs=[pl.BlockSpec((1024, 640), lambda i, j: (i, j))],
    out_specs=pl.BlockSpec((640, 1024), lambda i, j: (j, i)),
    out_shape=jax.ShapeDtypeStruct((6400, 4096), jnp.float32),
)


def kernel(inputs, table):
    t4 = _relayout_table(table.T)
    tview = t4.reshape(T4_ROWS * 4, EMBED)
    flat_idx = inputs.reshape(-1)
    out = _embed_lookup(flat_idx, tview)
    out2 = _transpose_out(out.reshape(4096, 6400))
    return out2.reshape(200, 32, 4096).transpose(2, 0, 1)


# speedup vs baseline: 2.5689x; 1.2364x over previous
import functools

import numpy as np

import jax
import jax.numpy as jnp
from jax import lax
from jax.experimental import pallas as pl
from jax.experimental.pallas import tpu as pltpu
from jax.experimental.pallas import tpu_sc as plsc

VOCAB = 1000000
EMBED = 32
B_TOTAL = 4096 * 200
NUM_CORES = 2
NUM_SUBCORES = 16
NW = NUM_CORES * NUM_SUBCORES
B_PER_W = B_TOTAL // NW
CHUNK = 1280
N_CHUNKS = B_PER_W // CHUNK

BV = 8192
SUB = BV // 512
NBLK = 123
T4_ROWS = NBLK * BV // 4

_mesh = plsc.VectorSubcoreMesh(core_axis_name="c", subcore_axis_name="s")


def _relayout_body(tt_ref, out_ref):
    t = tt_ref[...].T
    out_ref[...] = jnp.concatenate(
        [jnp.concatenate(
            [t[512 * q + 128 * c:512 * q + 128 * (c + 1), :]
             for c in range(4)], axis=1)
         for q in range(SUB)], axis=0)


_relayout_table = pl.pallas_call(
    _relayout_body,
    grid=(NBLK,),
    in_specs=[pl.BlockSpec((32, BV), lambda i: (0, i))],
    out_specs=pl.BlockSpec((BV // 4, 128), lambda i: (i, 0)),
    out_shape=jax.ShapeDtypeStruct((T4_ROWS, 128), jnp.float32),
)


@functools.partial(
    pl.kernel,
    mesh=_mesh,
    out_type=jax.ShapeDtypeStruct((B_TOTAL, EMBED), jnp.float32),
    scratch_types=[
        pltpu.VMEM((B_PER_W,), jnp.int32),
        pltpu.VMEM((CHUNK, EMBED), jnp.float32),
        pltpu.VMEM((CHUNK, EMBED), jnp.float32),
        pltpu.VMEM((CHUNK,), jnp.int32),
        pltpu.VMEM((CHUNK,), jnp.int32),
        pltpu.SemaphoreType.DMA,
        pltpu.SemaphoreType.DMA,
        pltpu.SemaphoreType.DMA,
        pltpu.SemaphoreType.DMA,
        pltpu.SemaphoreType.DMA,
        pltpu.SemaphoreType.DMA,
    ],
    compiler_params=pltpu.CompilerParams(use_tc_tiling_on_sc=False),
)
def _embed_lookup(idx_hbm, q_hbm, table_hbm, out_hbm,
                  idx_v, rows0, rows1, q0, q1,
                  sg0, sg1, so0, so1, sq0, sq1):
    wid = lax.axis_index("s") * NUM_CORES + lax.axis_index("c")
    base = wid * B_PER_W

    rows_v = (rows0, rows1)
    q_v = (q0, q1)
    sem_g = (sg0, sg1)
    sem_o = (so0, so1)
    sem_q = (sq0, sq1)

    def transform(j):
        def body(i, carry):
            off = j * CHUNK + i * 16
            v = idx_v[pl.ds(off, 16)]
            r = (v & -512) + ((v & 127) << 2) + ((v >> 7) & 3)
            idx_v[pl.ds(off, 16)] = r
            return carry
        lax.fori_loop(0, CHUNK // 16, body, 0)

    def gather(j, b):
        return pltpu.make_async_copy(
            table_hbm.at[idx_v.at[pl.ds(j * CHUNK, CHUNK)]], rows_v[b], sem_g[b])

    def q_dma(j, b):
        return pltpu.make_async_copy(
            q_hbm.at[pl.ds(base + j * CHUNK, CHUNK)], q_v[b], sem_q[b])

    def store(j, b):
        return pltpu.make_async_copy(
            rows_v[b], out_hbm.at[q_v[b]], sem_o[b])

    pltpu.sync_copy(idx_hbm.at[pl.ds(base, B_PER_W)], idx_v)

    transform(0)
    gather(0, 0).start()
    q_dma(0, 0).start()
    for j in range(N_CHUNKS):
        b = j & 1
        nb = b ^ 1
        if j + 1 < N_CHUNKS:
            if j >= 1:
                store(j - 1, nb).wait()
            transform(j + 1)
            gather(j + 1, nb).start()
            q_dma(j + 1, nb).start()
        gather(j, b).wait()
        q_dma(j, b).wait()
        store(j, b).start()
    store(N_CHUNKS - 2, 0).wait()
    store(N_CHUNKS - 1, 1).wait()


_FLAT = np.arange(4096 * 200)
_B = _FLAT // 200
_S = _FLAT % 200
_QSLOT = (_S // 4) * 16384 + _B * 4 + (_S % 4)


def _txp_body(in_ref, out_ref):
    for tc in range(50):
        g = in_ref[tc, :, :]
        out_ref[pl.ds(128 * tc, 128), :] = g.T


_transpose_out = pl.pallas_call(
    _txp_body,
    grid=(32,),
    compiler_params=pltpu.CompilerParams(vmem_limit_bytes=100 * 1024 * 1024),
    in_specs=[pl.BlockSpec((50, 128, 128), lambda i: (0, i, 0))],
    out_specs=pl.BlockSpec((6400, 128), lambda i: (0, i)),
    out_shape=jax.ShapeDtypeStruct((6400, 4096), jnp.float32),
)


def kernel(inputs, table):
    t4 = _relayout_table(table.T)
    tview = t4.reshape(T4_ROWS * 4, EMBED)
    flat_idx = inputs.reshape(-1)
    qslot = jnp.asarray(_QSLOT, dtype=jnp.int32)
    out = _embed_lookup(flat_idx, qslot, tview)
    out2 = _transpose_out(out.reshape(50, 4096, 128))
    return out2.reshape(200, 32, 4096).transpose(2, 0, 1)


# speedup vs baseline: 3.4708x; 1.3511x over previous
import functools

import numpy as np

import jax
import jax.numpy as jnp
from jax import lax
from jax.experimental import pallas as pl
from jax.experimental.pallas import tpu as pltpu
from jax.experimental.pallas import tpu_sc as plsc

VOCAB = 1000000
EMBED = 32
B_TOTAL = 4096 * 200
NUM_CORES = 2
NUM_SUBCORES = 16
NW = NUM_CORES * NUM_SUBCORES
B_PER_W = B_TOTAL // NW
CHUNK = 1280
N_CHUNKS = B_PER_W // CHUNK

BV = 8192
SUB = BV // 512
NBLK = 123
T4_ROWS = NBLK * BV // 4

_mesh = plsc.VectorSubcoreMesh(core_axis_name="c", subcore_axis_name="s")


def _relayout_body(tt_ref, out_ref):
    blk = tt_ref[...]
    for q in range(SUB):
        m = jnp.concatenate(
            [blk[:, 512 * q + 128 * c:512 * q + 128 * (c + 1)]
             for c in range(4)], axis=0)
        out_ref[pl.ds(128 * q, 128), :] = m.T


_relayout_table = pl.pallas_call(
    _relayout_body,
    grid=(NBLK,),
    in_specs=[pl.BlockSpec((32, BV), lambda i: (0, i))],
    out_specs=pl.BlockSpec((BV // 4, 128), lambda i: (i, 0)),
    out_shape=jax.ShapeDtypeStruct((T4_ROWS, 128), jnp.float32),
)


@functools.partial(
    pl.kernel,
    mesh=_mesh,
    out_type=jax.ShapeDtypeStruct((B_TOTAL, EMBED), jnp.float32),
    scratch_types=[
        pltpu.VMEM((B_PER_W,), jnp.int32),
        pltpu.VMEM((CHUNK, EMBED), jnp.float32),
        pltpu.VMEM((CHUNK, EMBED), jnp.float32),
        pltpu.VMEM((CHUNK,), jnp.int32),
        pltpu.VMEM((CHUNK,), jnp.int32),
        pltpu.SemaphoreType.DMA,
        pltpu.SemaphoreType.DMA,
        pltpu.SemaphoreType.DMA,
        pltpu.SemaphoreType.DMA,
        pltpu.SemaphoreType.DMA,
        pltpu.SemaphoreType.DMA,
    ],
    compiler_params=pltpu.CompilerParams(use_tc_tiling_on_sc=False),
)
def _embed_lookup(idx_hbm, q_hbm, table_hbm, out_hbm,
                  idx_v, rows0, rows1, q0, q1,
                  sg0, sg1, so0, so1, sq0, sq1):
    wid = lax.axis_index("s") * NUM_CORES + lax.axis_index("c")
    base = wid * B_PER_W

    rows_v = (rows0, rows1)
    q_v = (q0, q1)
    sem_g = (sg0, sg1)
    sem_o = (so0, so1)
    sem_q = (sq0, sq1)

    def transform(j):
        def body(i, carry):
            off = j * CHUNK + i * 16
            v = idx_v[pl.ds(off, 16)]
            r = (v & -512) + ((v & 127) << 2) + ((v >> 7) & 3)
            idx_v[pl.ds(off, 16)] = r
            return carry
        lax.fori_loop(0, CHUNK // 16, body, 0)

    def gather(j, b):
        return pltpu.make_async_copy(
            table_hbm.at[idx_v.at[pl.ds(j * CHUNK, CHUNK)]], rows_v[b], sem_g[b])

    def q_dma(j, b):
        return pltpu.make_async_copy(
            q_hbm.at[pl.ds(base + j * CHUNK, CHUNK)], q_v[b], sem_q[b])

    def store(j, b):
        return pltpu.make_async_copy(
            rows_v[b], out_hbm.at[q_v[b]], sem_o[b])

    pltpu.sync_copy(idx_hbm.at[pl.ds(base, B_PER_W)], idx_v)

    transform(0)
    gather(0, 0).start()
    q_dma(0, 0).start()
    for j in range(N_CHUNKS):
        b = j & 1
        nb = b ^ 1
        if j + 1 < N_CHUNKS:
            if j >= 1:
                store(j - 1, nb).wait()
            transform(j + 1)
            gather(j + 1, nb).start()
            q_dma(j + 1, nb).start()
        gather(j, b).wait()
        q_dma(j, b).wait()
        store(j, b).start()
    store(N_CHUNKS - 2, 0).wait()
    store(N_CHUNKS - 1, 1).wait()


_FLAT = np.arange(4096 * 200)
_B = _FLAT // 200
_S = _FLAT % 200
_QSLOT = (_S // 4) * 16384 + _B * 4 + (_S % 4)


def _txp_body(in_ref, out_ref):
    for tc in range(50):
        g = in_ref[tc, :, :]
        out_ref[pl.ds(128 * tc, 128), :] = g.T


_transpose_out = pl.pallas_call(
    _txp_body,
    grid=(32,),
    compiler_params=pltpu.CompilerParams(vmem_limit_bytes=100 * 1024 * 1024),
    in_specs=[pl.BlockSpec((50, 128, 128), lambda i: (0, i, 0))],
    out_specs=pl.BlockSpec((6400, 128), lambda i: (0, i)),
    out_shape=jax.ShapeDtypeStruct((6400, 4096), jnp.float32),
)


def kernel(inputs, table):
    t4 = _relayout_table(table.T)
    tview = t4.reshape(T4_ROWS * 4, EMBED)
    flat_idx = inputs.reshape(-1)
    qslot = jnp.asarray(_QSLOT, dtype=jnp.int32)
    out = _embed_lookup(flat_idx, qslot, tview)
    out2 = _transpose_out(out.reshape(50, 4096, 128))
    return out2.reshape(200, 32, 4096).transpose(2, 0, 1)
